# baseline (speedup 1.0000x reference)
import functools

import jax
import jax.numpy as jnp
from jax import lax
from jax.experimental import pallas as pl
from jax.experimental.pallas import tpu as pltpu
from jax.experimental.pallas import tpu_sc as plsc

VOCAB = 1000000
EMBED = 64
NCLS = 128
BATCH = 4096
SEQ = 50

NW = 32
N = BATCH * SEQ
ROWS_PER_W = N // NW
CHUNK = 128
NCHUNK = ROWS_PER_W // CHUNK

@functools.cache
def _build_sc_gather():
    mesh = plsc.VectorSubcoreMesh(core_axis_name="c", subcore_axis_name="s")

    @functools.partial(
        pl.kernel,
        mesh=mesh,
        out_type=jax.ShapeDtypeStruct((N, EMBED), jnp.float32),
        scratch_types=[
            pltpu.VMEM((NCHUNK, CHUNK), jnp.int32),
            pltpu.VMEM((2, CHUNK, EMBED), jnp.float32),
            pltpu.SemaphoreType.DMA,
            pltpu.SemaphoreType.DMA,
        ],
        compiler_params=pltpu.CompilerParams(use_tc_tiling_on_sc=False),
    )
    def _sc_gather(idx_hbm, table_hbm, out_hbm, idx_v, buf, gsem, wsem):
        wid = lax.axis_index("s") * 2 + lax.axis_index("c")
        base = wid * ROWS_PER_W
        pltpu.sync_copy(idx_hbm.at[wid], idx_v)

        def body(j, carry):
            pltpu.async_copy(table_hbm.at[idx_v.at[j]], buf.at[0], gsem).wait()
            pltpu.sync_copy(buf.at[0], out_hbm.at[pl.ds(base + j * CHUNK, CHUNK)])
            return carry

        lax.fori_loop(0, NCHUNK, body, 0)

    return _sc_gather


def _tc_body(emb_ref, w_ref, b_ref, out_ref):
    for g in range(emb_ref.shape[0]):
        e = emb_ref[g]
        logits = lax.dot_general(
            e, w_ref[...], (((1,), (1,)), ((), ())),
            preferred_element_type=jnp.float32)
        logits = logits + b_ref[...]
        m = jnp.max(logits, axis=0, keepdims=True)
        ex = jnp.exp(logits - m)
        out_ref[g] = ex / jnp.sum(ex, axis=0, keepdims=True)


def _dense_softmax(emb, W, b2, g=8):
    grid = BATCH // g
    return pl.pallas_call(
        _tc_body,
        grid=(grid,),
        in_specs=[
            pl.BlockSpec((g, SEQ, EMBED), lambda i: (i, 0, 0)),
            pl.BlockSpec((NCLS, EMBED), lambda i: (0, 0)),
            pl.BlockSpec((1, NCLS), lambda i: (0, 0)),
        ],
        out_specs=pl.BlockSpec((g, SEQ, NCLS), lambda i: (i, 0, 0)),
        out_shape=jax.ShapeDtypeStruct((BATCH, SEQ, NCLS), jnp.float32),
    )(emb, W, b2)


def kernel(text, table, W, b):
    idx = text.reshape(NW, NCHUNK, CHUNK).astype(jnp.int32)
    emb_flat = _build_sc_gather()(idx, table)
    emb = emb_flat.reshape(BATCH, SEQ, EMBED)
    return _dense_softmax(emb, W, b.reshape(1, NCLS))


# speedup vs baseline: 1.3050x; 1.3050x over previous
import functools

import jax
import jax.numpy as jnp
from jax import lax
from jax.experimental import pallas as pl
from jax.experimental.pallas import tpu as pltpu
from jax.experimental.pallas import tpu_sc as plsc

VOCAB = 1000000
EMBED = 64
NCLS = 128
BATCH = 4096
SEQ = 50

NW = 32
N = BATCH * SEQ
ROWS_PER_W = N // NW
CHUNK = 128
NCHUNK = ROWS_PER_W // CHUNK

@functools.cache
def _build_sc_gather():
    mesh = plsc.VectorSubcoreMesh(core_axis_name="c", subcore_axis_name="s")

    @functools.partial(
        pl.kernel,
        mesh=mesh,
        out_type=jax.ShapeDtypeStruct((N, EMBED), jnp.float32),
        scratch_types=[
            pltpu.VMEM((NCHUNK, CHUNK), jnp.int32),
            pltpu.VMEM((2, CHUNK, EMBED), jnp.float32),
            pltpu.SemaphoreType.DMA,
            pltpu.SemaphoreType.DMA,
        ],
        compiler_params=pltpu.CompilerParams(use_tc_tiling_on_sc=False),
    )
    def _sc_gather(idx_hbm, table_hbm, out_hbm, idx_v, buf, gsem, wsem):
        wid = lax.axis_index("s") * 2 + lax.axis_index("c")
        base = wid * ROWS_PER_W
        pltpu.sync_copy(idx_hbm.at[wid], idx_v)

        def body(j, carry):
            pltpu.async_copy(table_hbm.at[idx_v.at[j]], buf.at[0], gsem).wait()
            pltpu.sync_copy(buf.at[0], out_hbm.at[pl.ds(base + j * CHUNK, CHUNK)])
            return carry

        lax.fori_loop(0, NCHUNK, body, 0)

    return _sc_gather


HALF = SEQ // 2
WIDE = N // 2
G = 32
R = HALF * G


def _tc_body(x_ref, w2_ref, b2_ref, m_ref, mt_ref, out_ref):
    x = x_ref[...]
    logits = jnp.dot(x, w2_ref[...], preferred_element_type=jnp.float32)
    e = jnp.exp(logits + b2_ref[...])
    s = jnp.dot(m_ref[...], e, preferred_element_type=jnp.float32)
    d = s[:, :NCLS] + s[:, NCLS:]
    drep = jnp.dot(mt_ref[...], d, preferred_element_type=jnp.float32)
    rinv = 1.0 / drep
    o_l = e[:, :NCLS] * rinv
    o_r = e[:, NCLS:] * rinv
    for g in range(G):
        out_ref[g, :HALF, :] = o_l[g * HALF:(g + 1) * HALF, :]
        out_ref[g, HALF:, :] = o_r[g * HALF:(g + 1) * HALF, :]


def kernel(text, table, W, b):
    perm = jnp.stack([text[:, :HALF], text[:, HALF:]], axis=-1)
    idx = perm.reshape(NW, NCHUNK, CHUNK).astype(jnp.int32)
    emb_flat = _build_sc_gather()(idx, table)
    emb2 = emb_flat.reshape(WIDE, 2 * EMBED)

    wt = W.T
    w2 = jnp.zeros((2 * EMBED, 2 * NCLS), jnp.float32)
    w2 = w2.at[:EMBED, :NCLS].set(wt).at[EMBED:, NCLS:].set(wt)
    b2 = jnp.concatenate([b, b]).reshape(1, 2 * NCLS)
    gid = lax.broadcasted_iota(jnp.int32, (1, R), 1) // HALF
    m = (gid == lax.broadcasted_iota(jnp.int32, (G, 1), 0)).astype(jnp.float32)
    mt = m.T

    return pl.pallas_call(
        _tc_body,
        grid=(BATCH // G,),
        in_specs=[
            pl.BlockSpec((R, 2 * EMBED), lambda i: (i, 0)),
            pl.BlockSpec((2 * EMBED, 2 * NCLS), lambda i: (0, 0)),
            pl.BlockSpec((1, 2 * NCLS), lambda i: (0, 0)),
            pl.BlockSpec((G, R), lambda i: (0, 0)),
            pl.BlockSpec((R, G), lambda i: (0, 0)),
        ],
        out_specs=pl.BlockSpec((G, SEQ, NCLS), lambda i: (i, 0, 0)),
        out_shape=jax.ShapeDtypeStruct((BATCH, SEQ, NCLS), jnp.float32),
    )(emb2, w2, b2, m, mt)


# speedup vs baseline: 1.3728x; 1.0520x over previous
import functools

import jax
import jax.numpy as jnp
from jax import lax
from jax.experimental import pallas as pl
from jax.experimental.pallas import tpu as pltpu
from jax.experimental.pallas import tpu_sc as plsc

VOCAB = 1000000
EMBED = 64
NCLS = 128
BATCH = 4096
SEQ = 50

NW = 32
N = BATCH * SEQ
WROWS_PER_W = (N // 2) // NW
CHUNK = 128
NCHUNK = WROWS_PER_W // CHUNK

@functools.cache
def _build_sc_gather():
    mesh = plsc.VectorSubcoreMesh(core_axis_name="c", subcore_axis_name="s")

    @functools.partial(
        pl.kernel,
        mesh=mesh,
        out_type=jax.ShapeDtypeStruct((N // 2, 2 * EMBED), jnp.float32),
        scratch_types=[
            pltpu.VMEM((NCHUNK, CHUNK), jnp.int32),
            pltpu.VMEM((NCHUNK, CHUNK), jnp.int32),
            pltpu.VMEM((CHUNK, EMBED), jnp.float32),
            pltpu.VMEM((CHUNK, EMBED), jnp.float32),
            pltpu.SemaphoreType.DMA,
        ],
        compiler_params=pltpu.CompilerParams(use_tc_tiling_on_sc=False),
    )
    def _sc_gather(idxl_hbm, idxr_hbm, table_hbm, out_hbm,
                   idx_vl, idx_vr, bufl, bufr, gsem):
        wid = lax.axis_index("s") * 2 + lax.axis_index("c")
        wbase = wid * WROWS_PER_W
        pltpu.sync_copy(idxl_hbm.at[wid], idx_vl)
        pltpu.sync_copy(idxr_hbm.at[wid], idx_vr)

        def body(j, carry):
            cl = pltpu.async_copy(table_hbm.at[idx_vl.at[j]], bufl, gsem)
            cr = pltpu.async_copy(table_hbm.at[idx_vr.at[j]], bufr, gsem)
            cl.wait()
            cr.wait()
            r0 = wbase + j * CHUNK
            pltpu.sync_copy(bufl, out_hbm.at[pl.ds(r0, CHUNK), pl.ds(0, EMBED)])
            pltpu.sync_copy(bufr, out_hbm.at[pl.ds(r0, CHUNK), pl.ds(EMBED, EMBED)])
            return carry

        lax.fori_loop(0, NCHUNK, body, 0)

    return _sc_gather


HALF = SEQ // 2
WIDE = N // 2
G = 32
R = HALF * G


def _tc_body(x_ref, w2_ref, b2_ref, m_ref, mt_ref, out_ref):
    x = x_ref[...]
    logits = jnp.dot(x, w2_ref[...], preferred_element_type=jnp.float32)
    e = jnp.exp(logits + b2_ref[...])
    s = jnp.dot(m_ref[...], e, preferred_element_type=jnp.float32)
    d = s[:, :NCLS] + s[:, NCLS:]
    drep = jnp.dot(mt_ref[...], d, preferred_element_type=jnp.float32)
    rinv = 1.0 / drep
    o_l = e[:, :NCLS] * rinv
    o_r = e[:, NCLS:] * rinv
    for g in range(G):
        out_ref[g, :HALF, :] = o_l[g * HALF:(g + 1) * HALF, :]
        out_ref[g, HALF:, :] = o_r[g * HALF:(g + 1) * HALF, :]


def kernel(text, table, W, b):
    idx_l = text[:, :HALF].reshape(NW, NCHUNK, CHUNK).astype(jnp.int32)
    idx_r = text[:, HALF:].reshape(NW, NCHUNK, CHUNK).astype(jnp.int32)
    emb2 = _build_sc_gather()(idx_l, idx_r, table)

    wt = W.T
    w2 = jnp.zeros((2 * EMBED, 2 * NCLS), jnp.float32)
    w2 = w2.at[:EMBED, :NCLS].set(wt).at[EMBED:, NCLS:].set(wt)
    b2 = jnp.concatenate([b, b]).reshape(1, 2 * NCLS)
    gid = lax.broadcasted_iota(jnp.int32, (1, R), 1) // HALF
    m = (gid == lax.broadcasted_iota(jnp.int32, (G, 1), 0)).astype(jnp.float32)
    mt = m.T

    return pl.pallas_call(
        _tc_body,
        grid=(BATCH // G,),
        in_specs=[
            pl.BlockSpec((R, 2 * EMBED), lambda i: (i, 0)),
            pl.BlockSpec((2 * EMBED, 2 * NCLS), lambda i: (0, 0)),
            pl.BlockSpec((1, 2 * NCLS), lambda i: (0, 0)),
            pl.BlockSpec((G, R), lambda i: (0, 0)),
            pl.BlockSpec((R, G), lambda i: (0, 0)),
        ],
        out_specs=pl.BlockSpec((G, SEQ, NCLS), lambda i: (i, 0, 0)),
        out_shape=jax.ShapeDtypeStruct((BATCH, SEQ, NCLS), jnp.float32),
    )(emb2, w2, b2, m, mt)
